# baseline (speedup 1.0000x reference)
import functools

import jax
import jax.numpy as jnp
from jax import lax
from jax.experimental import pallas as pl
from jax.experimental.pallas import tpu as pltpu
from jax.experimental.pallas import tpu_sc as plsc

_B, _N, _F, _R = 8, 8192, 2048, 5
_LANES = 16
_NW = 32


def _score_body(x_ref, w_ref, o_ref):
    o_ref[...] = jnp.dot(x_ref[...], w_ref[...],
                         preferred_element_type=jnp.float32)


def _scores(feats_flat, w_col, blk):
    tot = feats_flat.shape[0]
    return pl.pallas_call(
        _score_body,
        grid=(tot // blk,),
        in_specs=[
            pl.BlockSpec((blk, _F), lambda i: (i, 0)),
            pl.BlockSpec((_F, 1), lambda i: (0, 0)),
        ],
        out_specs=pl.BlockSpec((blk, 1), lambda i: (i, 0)),
        out_shape=jax.ShapeDtypeStruct((tot, 1), jnp.float32),
    )(feats_flat, w_col)


def _select_body(scores_hbm, out_hbm, row_v, res_v):
    cid = lax.axis_index("c")
    sid = lax.axis_index("s")
    wid = sid * 2 + cid

    @pl.when(wid < _B)
    def _():
        pltpu.sync_copy(scores_hbm.at[wid], row_v)
        v0 = lax.sort(row_v[pl.ds(0, _LANES)])

        def body(i, carry):
            t_bot, t_top = carry
            s = lax.sort(row_v[pl.ds(i * _LANES, _LANES)])
            sd = lax.rev(s, (0,))
            t_top = lax.sort(jnp.maximum(t_top, sd))
            t_bot = lax.sort(jnp.minimum(t_bot, sd))
            return t_bot, t_top

        t_bot, t_top = lax.fori_loop(1, _N // _LANES, body, (v0, v0))
        res_v[pl.ds(0, _LANES)] = t_bot
        res_v[pl.ds(_LANES, _LANES)] = t_top
        pltpu.sync_copy(res_v, out_hbm.at[wid])


@functools.cache
def _select_kernel():
    return pl.kernel(
        _select_body,
        out_type=jax.ShapeDtypeStruct((_B, 2 * _LANES), jnp.float32),
        mesh=plsc.VectorSubcoreMesh(core_axis_name="c", subcore_axis_name="s"),
        scratch_types=[
            pltpu.VMEM((_N,), jnp.float32),
            pltpu.VMEM((2 * _LANES,), jnp.float32),
        ],
        compiler_params=pltpu.CompilerParams(needs_layout_passes=False),
    )


def _mlp_body(sel_ref, w1_ref, b1_ref, w2_ref, b2_ref, w3_ref, b3_ref,
              logits_ref, probs_ref):
    sel = sel_ref[...]
    x = jnp.concatenate([sel[:, 0:_R], sel[:, 32 - _R:32]], axis=1)

    def sigmoid(v):
        return 1.0 / (1.0 + jnp.exp(-v))

    h = sigmoid(jnp.dot(x, w1_ref[...], preferred_element_type=jnp.float32)
                + b1_ref[...])
    h = sigmoid(jnp.dot(h, w2_ref[...], preferred_element_type=jnp.float32)
                + b2_ref[...])
    logits = (jnp.dot(h, w3_ref[...], preferred_element_type=jnp.float32)
              + b3_ref[...])
    logits_ref[...] = logits
    probs_ref[...] = sigmoid(logits)


def _mlp(sel, W1, b1, W2, b2, W3, b3):
    return pl.pallas_call(
        _mlp_body,
        out_shape=(
            jax.ShapeDtypeStruct((_B, 1), jnp.float32),
            jax.ShapeDtypeStruct((_B, 1), jnp.float32),
        ),
    )(sel, W1, b1.reshape(1, -1), W2, b2.reshape(1, -1), W3,
      b3.reshape(1, -1))


def kernel(feats, w_conv, W1, b1, W2, b2, W3, b3):
    feats_flat = feats.reshape(_B * _N, _F)
    scores = _scores(feats_flat, w_conv.reshape(_F, 1), blk=512)
    sel = _select_kernel()(scores.reshape(_B, _N))
    logits, probs = _mlp(sel, W1, b1, W2, b2, W3, b3)
    return logits, probs


# speedup vs baseline: 1.1252x; 1.1252x over previous
import functools

import jax
import jax.numpy as jnp
from jax import lax
from jax.experimental import pallas as pl
from jax.experimental.pallas import tpu as pltpu
from jax.experimental.pallas import tpu_sc as plsc

_B, _N, _F, _R = 8, 8192, 2048, 5
_LANES = 16
_NW = 32


def _score_body(x_ref, w_ref, o_ref):
    o_ref[...] = lax.dot_general(
        w_ref[...], x_ref[...], (((1,), (1,)), ((), ())),
        preferred_element_type=jnp.float32)


def _scores(feats_flat, w_row, blk):
    tot = feats_flat.shape[0]
    return pl.pallas_call(
        _score_body,
        grid=(tot // blk,),
        in_specs=[
            pl.BlockSpec((blk, _F), lambda i: (i, 0)),
            pl.BlockSpec((1, _F), lambda i: (0, 0)),
        ],
        out_specs=pl.BlockSpec((1, blk), lambda i: (0, i)),
        out_shape=jax.ShapeDtypeStruct((1, tot), jnp.float32),
    )(feats_flat, w_row)


def _select_body(scores_hbm, out_hbm, row_v, res_v):
    cid = lax.axis_index("c")
    sid = lax.axis_index("s")
    wid = sid * 2 + cid

    @pl.when(wid < _B)
    def _():
        pltpu.sync_copy(scores_hbm.at[wid], row_v)
        v0 = lax.sort(row_v[pl.ds(0, _LANES)])

        def body(i, carry):
            t_bot, t_top = carry
            s = lax.sort(row_v[pl.ds(i * _LANES, _LANES)])
            sd = lax.rev(s, (0,))
            t_top = lax.sort(jnp.maximum(t_top, sd))
            t_bot = lax.sort(jnp.minimum(t_bot, sd))
            return t_bot, t_top

        t_bot, t_top = lax.fori_loop(1, _N // _LANES, body, (v0, v0))
        res_v[pl.ds(0, _LANES)] = t_bot
        res_v[pl.ds(_LANES, _LANES)] = t_top
        pltpu.sync_copy(res_v, out_hbm.at[wid])


@functools.cache
def _select_kernel():
    return pl.kernel(
        _select_body,
        out_type=jax.ShapeDtypeStruct((_B, 2 * _LANES), jnp.float32),
        mesh=plsc.VectorSubcoreMesh(core_axis_name="c", subcore_axis_name="s"),
        scratch_types=[
            pltpu.VMEM((_N,), jnp.float32),
            pltpu.VMEM((2 * _LANES,), jnp.float32),
        ],
        compiler_params=pltpu.CompilerParams(needs_layout_passes=False),
    )


def _mlp_body(sel_ref, w1_ref, b1_ref, w2_ref, b2_ref, w3_ref, b3_ref,
              logits_ref, probs_ref):
    sel = sel_ref[...]
    x = jnp.concatenate([sel[:, 0:_R], sel[:, 32 - _R:32]], axis=1)

    def sigmoid(v):
        return 1.0 / (1.0 + jnp.exp(-v))

    h = sigmoid(jnp.dot(x, w1_ref[...], preferred_element_type=jnp.float32)
                + b1_ref[...])
    h = sigmoid(jnp.dot(h, w2_ref[...], preferred_element_type=jnp.float32)
                + b2_ref[...])
    logits = (jnp.dot(h, w3_ref[...], preferred_element_type=jnp.float32)
              + b3_ref[...])
    logits_ref[...] = logits
    probs_ref[...] = sigmoid(logits)


def _mlp(sel, W1, b1, W2, b2, W3, b3):
    return pl.pallas_call(
        _mlp_body,
        out_shape=(
            jax.ShapeDtypeStruct((_B, 1), jnp.float32),
            jax.ShapeDtypeStruct((_B, 1), jnp.float32),
        ),
    )(sel, W1, b1.reshape(1, -1), W2, b2.reshape(1, -1), W3,
      b3.reshape(1, -1))


def kernel(feats, w_conv, W1, b1, W2, b2, W3, b3):
    feats_flat = feats.reshape(_B * _N, _F)
    scores = _scores(feats_flat, w_conv.reshape(1, _F), blk=512)
    sel = _select_kernel()(scores.reshape(_B, _N))
    logits, probs = _mlp(sel, W1, b1, W2, b2, W3, b3)
    return logits, probs
